# Initial kernel scaffold; baseline (speedup 1.0000x reference)
import jax
import jax.numpy as jnp
from jax.experimental import pallas as pl


def kernel(x, centriods, assignments, rowwise_norms, columnwise_norms):
    raise NotImplementedError("write your pallas kernel here")


# speedup vs baseline: 19.8303x; 19.8303x over previous
import functools

import jax
import jax.numpy as jnp
from jax import lax
from jax.experimental import pallas as pl
from jax.experimental.pallas import tpu as pltpu
from jax.experimental.pallas import tpu_sc as plsc

_N_OUT = 4096
_N_IN = 4096
_D = 4
_K = 256
_TOKENS = 8192
_GROUPS = _N_IN // _D
_WORDS = _N_IN // 2

_NC, _NS = 2, 16
_NW = _NC * _NS
_ROWS_PER_W = _N_OUT // _NW
_R_CHUNK = 8
_N_CHUNKS = _ROWS_PER_W // _R_CHUNK


def _sc_gather(a_flat, p_tab):
    mesh = plsc.VectorSubcoreMesh(core_axis_name="c", subcore_axis_name="s")

    @functools.partial(
        pl.kernel,
        mesh=mesh,
        out_type=jax.ShapeDtypeStruct((_N_OUT * _WORDS,), jnp.int32),
        scratch_types=[
            pltpu.VMEM((2 * _K,), jnp.int32),
            pltpu.VMEM((_R_CHUNK * _GROUPS,), jnp.int32),
            pltpu.VMEM((_R_CHUNK * _WORDS,), jnp.int32),
        ],
        compiler_params=pltpu.CompilerParams(needs_layout_passes=False),
    )
    def k(a_hbm, p_hbm, out_hbm, p_v, a_v, out_v):
        wid = lax.axis_index("s") * _NC + lax.axis_index("c")
        pltpu.sync_copy(p_hbm, p_v)
        iota = lax.iota(jnp.int32, 16)
        iota_half = iota >> 1
        parity = iota & 1

        def chunk_body(ci, _):
            row0 = wid * _ROWS_PER_W + ci * _R_CHUNK
            a_start = pl.multiple_of(row0 * _GROUPS, 8)
            pltpu.sync_copy(a_hbm.at[pl.ds(a_start, _R_CHUNK * _GROUPS)], a_v)

            def word_body(t, _):
                gidx = jnp.full((16,), t * 8, jnp.int32) + iota_half
                a_vec = plsc.load_gather(a_v, [gidx])
                pidx = (a_vec << 1) | parity
                w_vec = plsc.load_gather(p_v, [pidx])
                out_v[pl.ds(t * 16, 16)] = w_vec
                return _

            lax.fori_loop(0, _R_CHUNK * _WORDS // 16, word_body, None)
            o_start = pl.multiple_of(row0 * _WORDS, 8)
            pltpu.sync_copy(out_v, out_hbm.at[pl.ds(o_start, _R_CHUNK * _WORDS)])
            return _

        lax.fori_loop(0, _N_CHUNKS, chunk_body, None)

    return k(a_flat, p_tab)


_BT_X = 512


def _precast_body(x_ref, r_ref, o_ref):
    o_ref[...] = (x_ref[...] * r_ref[...]).astype(jnp.bfloat16)


def _precast(x, row2d):
    return pl.pallas_call(
        _precast_body,
        grid=(_TOKENS // _BT_X,),
        in_specs=[
            pl.BlockSpec((_BT_X, _N_IN), lambda t: (t, 0)),
            pl.BlockSpec((1, _N_IN), lambda t: (0, 0)),
        ],
        out_specs=pl.BlockSpec((_BT_X, _N_IN), lambda t: (t, 0)),
        out_shape=jax.ShapeDtypeStruct((_TOKENS, _N_IN), jnp.bfloat16),
    )(x, row2d)


_BT, _BI, _BK = 2048, 2048, 512
_NT, _NI, _NK = _TOKENS // _BT, _N_OUT // _BI, _N_IN // _BK


def _matmul_body(x_ref, w_ref, c_ref, o_ref):
    k = pl.program_id(2)

    @pl.when(k == 0)
    def _init():
        o_ref[...] = jnp.zeros_like(o_ref)

    o_ref[...] += lax.dot_general(
        x_ref[...], w_ref[...],
        (((1,), (1,)), ((), ())),
        preferred_element_type=jnp.float32,
    )

    @pl.when(k == _NK - 1)
    def _fin():
        o_ref[...] *= c_ref[...]


def _matmul(xb, w_bf, col2d):
    return pl.pallas_call(
        _matmul_body,
        grid=(_NT, _NI, _NK),
        in_specs=[
            pl.BlockSpec((_BT, _BK), lambda t, i, k: (t, k)),
            pl.BlockSpec((_BI, _BK), lambda t, i, k: (i, k)),
            pl.BlockSpec((1, _BI), lambda t, i, k: (0, i)),
        ],
        out_specs=pl.BlockSpec((_BT, _BI), lambda t, i, k: (t, i)),
        out_shape=jax.ShapeDtypeStruct((_TOKENS, _N_OUT), jnp.float32),
        compiler_params=pltpu.CompilerParams(
            dimension_semantics=("parallel", "parallel", "arbitrary"),
        ),
    )(xb, w_bf, col2d)


def kernel(x, centriods, assignments, rowwise_norms, columnwise_norms):
    a = assignments.astype(jnp.int32)
    cb = centriods.astype(jnp.bfloat16).reshape(_K, _D // 2, 2)
    p_tab = lax.bitcast_convert_type(cb, jnp.int32).reshape(-1)
    w_words = _sc_gather(a, p_tab)
    w_bf = lax.bitcast_convert_type(
        w_words.reshape(_N_OUT, _WORDS), jnp.bfloat16
    ).reshape(_N_OUT, _N_IN)
    xb = _precast(x, rowwise_norms.reshape(1, _N_IN))
    return _matmul(xb, w_bf, columnwise_norms.reshape(1, _N_OUT))


# speedup vs baseline: 24.1379x; 1.2172x over previous
import functools

import jax
import jax.numpy as jnp
from jax import lax
from jax.experimental import pallas as pl
from jax.experimental.pallas import tpu as pltpu
from jax.experimental.pallas import tpu_sc as plsc

_N_OUT = 4096
_N_IN = 4096
_D = 4
_K = 256
_TOKENS = 8192
_GROUPS = _N_IN // _D
_WORDS = _N_IN // 2

_NC, _NS = 2, 16
_NW = _NC * _NS
_ROWS_PER_W = _N_OUT // _NW
_R_CHUNK = 8
_N_CHUNKS = _ROWS_PER_W // _R_CHUNK


def _sc_gather(a_flat, p_tab):
    mesh = plsc.VectorSubcoreMesh(core_axis_name="c", subcore_axis_name="s")

    @functools.partial(
        pl.kernel,
        mesh=mesh,
        out_type=jax.ShapeDtypeStruct((_N_OUT * _WORDS,), jnp.int32),
        scratch_types=[
            pltpu.VMEM((2 * _K,), jnp.int32),
            pltpu.VMEM((_R_CHUNK * _GROUPS,), jnp.int32),
            pltpu.VMEM((_R_CHUNK * _WORDS,), jnp.int32),
        ],
        compiler_params=pltpu.CompilerParams(needs_layout_passes=False),
    )
    def k(a_hbm, p_hbm, out_hbm, p_v, a_v, out_v):
        wid = lax.axis_index("s") * _NC + lax.axis_index("c")
        pltpu.sync_copy(p_hbm, p_v)
        iota = lax.iota(jnp.int32, 16)
        iota_half = iota >> 1
        parity = iota & 1

        def chunk_body(ci, _):
            row0 = wid * _ROWS_PER_W + ci * _R_CHUNK
            a_start = pl.multiple_of(row0 * _GROUPS, 8)
            pltpu.sync_copy(a_hbm.at[pl.ds(a_start, _R_CHUNK * _GROUPS)], a_v)

            @plsc.parallel_loop(0, _R_CHUNK * _WORDS // 16, unroll=8)
            def word_body(t):
                gidx = jnp.full((16,), t * 8, jnp.int32) + iota_half
                a_vec = plsc.load_gather(a_v, [gidx])
                pidx = (a_vec << 1) | parity
                w_vec = plsc.load_gather(p_v, [pidx])
                out_v[pl.ds(t * 16, 16)] = w_vec

            o_start = pl.multiple_of(row0 * _WORDS, 8)
            pltpu.sync_copy(out_v, out_hbm.at[pl.ds(o_start, _R_CHUNK * _WORDS)])
            return _

        lax.fori_loop(0, _N_CHUNKS, chunk_body, None)

    return k(a_flat, p_tab)


_BT_X = 512


def _precast_body(x_ref, r_ref, o_ref):
    o_ref[...] = (x_ref[...] * r_ref[...]).astype(jnp.bfloat16)


def _precast(x, row2d):
    return pl.pallas_call(
        _precast_body,
        grid=(_TOKENS // _BT_X,),
        in_specs=[
            pl.BlockSpec((_BT_X, _N_IN), lambda t: (t, 0)),
            pl.BlockSpec((1, _N_IN), lambda t: (0, 0)),
        ],
        out_specs=pl.BlockSpec((_BT_X, _N_IN), lambda t: (t, 0)),
        out_shape=jax.ShapeDtypeStruct((_TOKENS, _N_IN), jnp.bfloat16),
    )(x, row2d)


_BT, _BI, _BK = 2048, 2048, 512
_NT, _NI, _NK = _TOKENS // _BT, _N_OUT // _BI, _N_IN // _BK


def _matmul_body(x_ref, w_ref, c_ref, o_ref):
    k = pl.program_id(2)

    @pl.when(k == 0)
    def _init():
        o_ref[...] = jnp.zeros_like(o_ref)

    o_ref[...] += lax.dot_general(
        x_ref[...], w_ref[...],
        (((1,), (1,)), ((), ())),
        preferred_element_type=jnp.float32,
    )

    @pl.when(k == _NK - 1)
    def _fin():
        o_ref[...] *= c_ref[...]


def _matmul(xb, w_bf, col2d):
    return pl.pallas_call(
        _matmul_body,
        grid=(_NT, _NI, _NK),
        in_specs=[
            pl.BlockSpec((_BT, _BK), lambda t, i, k: (t, k)),
            pl.BlockSpec((_BI, _BK), lambda t, i, k: (i, k)),
            pl.BlockSpec((1, _BI), lambda t, i, k: (0, i)),
        ],
        out_specs=pl.BlockSpec((_BT, _BI), lambda t, i, k: (t, i)),
        out_shape=jax.ShapeDtypeStruct((_TOKENS, _N_OUT), jnp.float32),
        compiler_params=pltpu.CompilerParams(
            dimension_semantics=("parallel", "parallel", "arbitrary"),
        ),
    )(xb, w_bf, col2d)


def kernel(x, centriods, assignments, rowwise_norms, columnwise_norms):
    a = assignments.astype(jnp.int32)
    cb = centriods.astype(jnp.bfloat16).reshape(_K, _D // 2, 2)
    p_tab = lax.bitcast_convert_type(cb, jnp.int32).reshape(-1)
    w_words = _sc_gather(a, p_tab)
    w_bf = lax.bitcast_convert_type(
        w_words.reshape(_N_OUT, _WORDS), jnp.bfloat16
    ).reshape(_N_OUT, _N_IN)
    xb = _precast(x, rowwise_norms.reshape(1, _N_IN))
    return _matmul(xb, w_bf, columnwise_norms.reshape(1, _N_OUT))


# speedup vs baseline: 42.7313x; 1.7703x over previous
import functools

import jax
import jax.numpy as jnp
from jax import lax
from jax.experimental import pallas as pl
from jax.experimental.pallas import tpu as pltpu
from jax.experimental.pallas import tpu_sc as plsc

_N_OUT = 4096
_N_IN = 4096
_D = 4
_K = 256
_TOKENS = 8192
_GROUPS = _N_IN // _D
_RP = _N_OUT // 2

_NC, _NS = 2, 16
_NW = _NC * _NS
_RP_PER_W = _RP // _NW
_RP_CHUNK = 8
_N_CHUNKS = _RP_PER_W // _RP_CHUNK


def _sc_gather(a_flat, p_tab):
    mesh = plsc.VectorSubcoreMesh(core_axis_name="c", subcore_axis_name="s")

    @functools.partial(
        pl.kernel,
        mesh=mesh,
        out_type=jax.ShapeDtypeStruct((_RP, _N_IN), jnp.int32),
        scratch_types=[
            pltpu.VMEM((_K * _D,), jnp.int32),
            pltpu.VMEM((2 * _RP_CHUNK * _GROUPS,), jnp.int32),
            pltpu.VMEM((_RP_CHUNK, _N_IN), jnp.int32),
        ],
        compiler_params=pltpu.CompilerParams(needs_layout_passes=False),
    )
    def k(a_hbm, p_hbm, out_hbm, p_v, a_v, out_v):
        wid = lax.axis_index("s") * _NC + lax.axis_index("c")
        pltpu.sync_copy(p_hbm, p_v)
        iota = lax.iota(jnp.int32, 16)
        iota_q = iota >> 2
        dmask = iota & 3

        def chunk_body(ci, _):
            rp0 = wid * _RP_PER_W + ci * _RP_CHUNK
            a_start = pl.multiple_of(2 * rp0 * _GROUPS, 8)
            pltpu.sync_copy(
                a_hbm.at[pl.ds(a_start, 2 * _RP_CHUNK * _GROUPS)], a_v
            )

            for rp_l in range(_RP_CHUNK):
                base0 = 2 * rp_l * _GROUPS

                @plsc.parallel_loop(0, _N_IN // 16, unroll=8)
                def word_body(t):
                    g0 = jnp.full((16,), t * 4 + base0, jnp.int32) + iota_q
                    a0 = plsc.load_gather(a_v, [g0])
                    a1 = plsc.load_gather(a_v, [g0 + _GROUPS])
                    w0 = plsc.load_gather(p_v, [(a0 << 2) | dmask])
                    w1 = plsc.load_gather(p_v, [(a1 << 2) | dmask])
                    out_v[rp_l, pl.ds(t * 16, 16)] = w0 | (w1 << 16)

            pltpu.sync_copy(out_v, out_hbm.at[pl.ds(pl.multiple_of(rp0, 8), _RP_CHUNK)])
            return _

        lax.fori_loop(0, _N_CHUNKS, chunk_body, None)

    return k(a_flat, p_tab)


_BT, _BI, _BK = 2048, 2048, 512
_NT, _NI, _NK = _TOKENS // _BT, _N_OUT // _BI, _N_IN // _BK


def _matmul_body(x_ref, w_ref, r_ref, c_ref, o_ref):
    k = pl.program_id(2)

    @pl.when(k == 0)
    def _init():
        o_ref[...] = jnp.zeros_like(o_ref)

    xb = (x_ref[...] * r_ref[...]).astype(jnp.bfloat16)
    wb = pltpu.bitcast(w_ref[...], jnp.bfloat16)
    o_ref[...] += lax.dot_general(
        xb, wb,
        (((1,), (1,)), ((), ())),
        preferred_element_type=jnp.float32,
    )

    @pl.when(k == _NK - 1)
    def _fin():
        o_ref[...] *= c_ref[...]


def _matmul(x, w_words, row2d, col2d):
    return pl.pallas_call(
        _matmul_body,
        grid=(_NT, _NI, _NK),
        in_specs=[
            pl.BlockSpec((_BT, _BK), lambda t, i, k: (t, k)),
            pl.BlockSpec((_BI // 2, _BK), lambda t, i, k: (i, k)),
            pl.BlockSpec((1, _BK), lambda t, i, k: (0, k)),
            pl.BlockSpec((1, _BI), lambda t, i, k: (0, i)),
        ],
        out_specs=pl.BlockSpec((_BT, _BI), lambda t, i, k: (t, i)),
        out_shape=jax.ShapeDtypeStruct((_TOKENS, _N_OUT), jnp.float32),
        compiler_params=pltpu.CompilerParams(
            dimension_semantics=("parallel", "parallel", "arbitrary"),
        ),
    )(x, w_words, row2d, col2d)


def kernel(x, centriods, assignments, rowwise_norms, columnwise_norms):
    a = assignments.astype(jnp.int32)
    cb16 = lax.bitcast_convert_type(centriods.astype(jnp.bfloat16), jnp.uint16)
    p_tab = cb16.astype(jnp.int32).reshape(_K * _D)
    w_words = _sc_gather(a, p_tab)
    return _matmul(
        x, w_words,
        rowwise_norms.reshape(1, _N_IN),
        columnwise_norms.reshape(1, _N_OUT),
    )


# speedup vs baseline: 43.3513x; 1.0145x over previous
import functools

import jax
import jax.numpy as jnp
from jax import lax
from jax.experimental import pallas as pl
from jax.experimental.pallas import tpu as pltpu
from jax.experimental.pallas import tpu_sc as plsc

_N_OUT = 4096
_N_IN = 4096
_D = 4
_K = 256
_TOKENS = 8192
_GROUPS = _N_IN // _D
_RP = _N_OUT // 2

_NC, _NS = 2, 16
_NW = _NC * _NS
_RP_PER_W = _RP // _NW
_RP_CHUNK = 8
_N_CHUNKS = _RP_PER_W // _RP_CHUNK


def _sc_gather(a_flat, p_tab):
    mesh = plsc.VectorSubcoreMesh(core_axis_name="c", subcore_axis_name="s")

    @functools.partial(
        pl.kernel,
        mesh=mesh,
        out_type=jax.ShapeDtypeStruct((_RP, _N_IN), jnp.int32),
        scratch_types=[
            pltpu.VMEM((_K * _D,), jnp.int32),
            pltpu.VMEM((2 * _RP_CHUNK * _GROUPS,), jnp.int32),
            pltpu.VMEM((_RP_CHUNK, _N_IN), jnp.int32),
        ],
        compiler_params=pltpu.CompilerParams(needs_layout_passes=False),
    )
    def k(a_hbm, p_hbm, out_hbm, p_v, a_v, out_v):
        wid = lax.axis_index("s") * _NC + lax.axis_index("c")
        pltpu.sync_copy(p_hbm, p_v)
        iota = lax.iota(jnp.int32, 16)
        iota_q = iota >> 2
        dmask = iota & 3

        def chunk_body(ci, _):
            rp0 = wid * _RP_PER_W + ci * _RP_CHUNK
            a_start = pl.multiple_of(2 * rp0 * _GROUPS, 8)
            pltpu.sync_copy(
                a_hbm.at[pl.ds(a_start, 2 * _RP_CHUNK * _GROUPS)], a_v
            )

            for rp_l in range(_RP_CHUNK):
                base0 = 2 * rp_l * _GROUPS

                @plsc.parallel_loop(0, _N_IN // 16, unroll=8)
                def word_body(t):
                    g0 = jnp.full((16,), t * 4 + base0, jnp.int32) + iota_q
                    a0 = plsc.load_gather(a_v, [g0])
                    a1 = plsc.load_gather(a_v, [g0 + _GROUPS])
                    w0 = plsc.load_gather(p_v, [(a0 << 2) | dmask])
                    w1 = plsc.load_gather(p_v, [(a1 << 2) | dmask])
                    out_v[rp_l, pl.ds(t * 16, 16)] = w0 | (w1 << 16)

            pltpu.sync_copy(out_v, out_hbm.at[pl.ds(pl.multiple_of(rp0, 8), _RP_CHUNK)])
            return _

        lax.fori_loop(0, _N_CHUNKS, chunk_body, None)

    return k(a_flat, p_tab)


_BT_X = 1024


def _precast_body(x_ref, r_ref, o_ref):
    o_ref[...] = (x_ref[...] * r_ref[...]).astype(jnp.bfloat16)


def _precast(x, row2d):
    return pl.pallas_call(
        _precast_body,
        grid=(_TOKENS // _BT_X,),
        in_specs=[
            pl.BlockSpec((_BT_X, _N_IN), lambda t: (t, 0)),
            pl.BlockSpec((1, _N_IN), lambda t: (0, 0)),
        ],
        out_specs=pl.BlockSpec((_BT_X, _N_IN), lambda t: (t, 0)),
        out_shape=jax.ShapeDtypeStruct((_TOKENS, _N_IN), jnp.bfloat16),
    )(x, row2d)


_BT, _BI, _BK = 2048, 2048, 1024
_NT, _NI, _NK = _TOKENS // _BT, _N_OUT // _BI, _N_IN // _BK


def _matmul_body(x_ref, w_ref, c_ref, o_ref):
    k = pl.program_id(2)

    @pl.when(k == 0)
    def _init():
        o_ref[...] = jnp.zeros_like(o_ref)

    wb = pltpu.bitcast(w_ref[...], jnp.bfloat16)
    o_ref[...] += lax.dot_general(
        x_ref[...], wb,
        (((1,), (1,)), ((), ())),
        preferred_element_type=jnp.float32,
    )

    @pl.when(k == _NK - 1)
    def _fin():
        o_ref[...] *= c_ref[...]


def _matmul(xb, w_words, col2d):
    return pl.pallas_call(
        _matmul_body,
        grid=(_NT, _NI, _NK),
        in_specs=[
            pl.BlockSpec((_BT, _BK), lambda t, i, k: (t, k)),
            pl.BlockSpec((_BI // 2, _BK), lambda t, i, k: (i, k)),
            pl.BlockSpec((1, _BI), lambda t, i, k: (0, i)),
        ],
        out_specs=pl.BlockSpec((_BT, _BI), lambda t, i, k: (t, i)),
        out_shape=jax.ShapeDtypeStruct((_TOKENS, _N_OUT), jnp.float32),
        compiler_params=pltpu.CompilerParams(
            dimension_semantics=("parallel", "parallel", "arbitrary"),
        ),
    )(xb, w_words, col2d)


def kernel(x, centriods, assignments, rowwise_norms, columnwise_norms):
    a = assignments.astype(jnp.int32)
    cb16 = lax.bitcast_convert_type(centriods.astype(jnp.bfloat16), jnp.uint16)
    p_tab = cb16.astype(jnp.int32).reshape(_K * _D)
    w_words = _sc_gather(a, p_tab)
    xb = _precast(x, rowwise_norms.reshape(1, _N_IN))
    return _matmul(xb, w_words, columnwise_norms.reshape(1, _N_OUT))
